# baseline (speedup 1.0000x reference)
import functools
import math

import jax
import jax.numpy as jnp
from jax import lax
from jax.experimental import pallas as pl
from jax.experimental.pallas import tpu as pltpu
from jax.experimental.pallas import tpu_sc as plsc

N = 10000
E = 320000
DIM = 128
T = 4
R = 8
H = 8
DK = 16

NC = 2
NS = 16
NW = NC * NS
CA = 128
NCHA = 2 * (-(-E // (NW * CA * 2)))
EPW = NCHA * CA
EP = NW * EPW
EPX = (NW * NCHA + 2) * CA
CB = 64
NCHB = EPW // CB
ACC_W = 144
NPAD = N + 16
ROWS_PT = N // NS
BE = 2048

_mesh = plsc.VectorSubcoreMesh(core_axis_name="c", subcore_axis_name="s")
_sc_params = pltpu.CompilerParams(use_tc_tiling_on_sc=False,
                                  needs_layout_passes=False)


def _proj_body(x_ref, t_ref, wq_ref, bq_ref, wk_ref, bk_ref, wv_ref, bv_ref,
               q_ref, kv_ref):
    x = x_ref[...]
    t = t_ref[...]
    for out_ref, col, w_ref, b_ref in ((q_ref, 0, wq_ref, bq_ref),
                                       (kv_ref, 0, wk_ref, bk_ref),
                                       (kv_ref, 1, wv_ref, bv_ref)):
        acc = jnp.zeros(x.shape, jnp.float32)
        for tt in range(T):
            sel = (t == tt).astype(jnp.float32)
            y = jnp.dot(x, w_ref[tt], preferred_element_type=jnp.float32)
            acc = acc + sel * (y + b_ref[tt:tt + 1, :])
        out_ref[:, col * DIM:(col + 1) * DIM] = acc


def _project(node_inp, node_type2d, Wq, bq, Wk, bk, Wv, bv):
    B = 1000
    grid = (N // B,)
    row = pl.BlockSpec((B, DIM), lambda i: (i, 0))
    row2 = pl.BlockSpec((B, 2 * DIM), lambda i: (i, 0))
    tspec = pl.BlockSpec((B, 1), lambda i: (i, 0))
    wspec = pl.BlockSpec((T, DIM, DIM), lambda i: (0, 0, 0))
    bspec = pl.BlockSpec((T, DIM), lambda i: (0, 0))
    return pl.pallas_call(
        _proj_body,
        grid=grid,
        in_specs=[row, tspec, wspec, bspec, wspec, bspec, wspec, bspec],
        out_specs=[row, row2],
        out_shape=[jax.ShapeDtypeStruct((N, DIM), jnp.float32),
                   jax.ShapeDtypeStruct((N, 2 * DIM), jnp.float32)],
    )(node_inp, node_type2d, Wq, bq, Wk, bk, Wv, bv)


@functools.partial(
    pl.kernel,
    mesh=_mesh,
    compiler_params=_sc_params,
    out_type=[jax.ShapeDtypeStruct((EP, DIM), jnp.bfloat16),
              jax.ShapeDtypeStruct((EP, 2 * DIM), jnp.bfloat16)],
    scratch_types=[
        pltpu.VMEM((2, 2, CA), jnp.int32),
        pltpu.VMEM((2, CA), jnp.int32),
        pltpu.VMEM((2, CA), jnp.int32),
        pltpu.VMEM((2, CA, DIM), jnp.bfloat16),
        pltpu.VMEM((2, CA, 2 * DIM), jnp.bfloat16),
        pltpu.SemaphoreType.DMA,
        pltpu.SemaphoreType.DMA,
        pltpu.SemaphoreType.DMA,
        pltpu.SemaphoreType.DMA,
        pltpu.SemaphoreType.DMA,
        pltpu.SemaphoreType.DMA,
    ],
)
def _gather_kernel(e2_hbm, q_hbm, kv_hbm, qg_hbm, kvg_hbm,
                   e2_v, src_v, dst_v, q_rows, kv_rows,
                   semi0, semi1, semq0, semq1, semk0, semk1):
    c = lax.axis_index("c")
    s = lax.axis_index("s")
    wid = c * NS + s
    base0 = wid * NCHA
    semi = (semi0, semi1)
    semq = (semq0, semq1)
    semk = (semk0, semk1)

    def _unpack(b):
        for i in range(CA // 16):
            sl = pl.ds(i * 16, 16)
            src_v[b, sl] = e2_v[b, 0, sl]
            dst_v[b, sl] = e2_v[b, 1, sl]

    def _issue_idx(j, b):
        return pltpu.async_copy(e2_hbm.at[base0 + j], e2_v.at[b], semi[b])

    def _issue_gathers(b):
        pltpu.async_copy(q_hbm.at[dst_v.at[b]], q_rows.at[b], semq[b])
        pltpu.async_copy(kv_hbm.at[src_v.at[b]], kv_rows.at[b], semk[b])

    _issue_idx(0, 0).wait()
    _unpack(0)
    _issue_gathers(0)
    _issue_idx(1, 1)

    @pl.loop(0, NCHA, step=2)
    def _(j):
        for b in range(2):
            pltpu.make_async_copy(e2_hbm.at[0], e2_v.at[1 - b],
                                  semi[1 - b]).wait()
            _unpack(1 - b)
            _issue_gathers(1 - b)
            _issue_idx(j + b + 2, b)
            pltpu.make_async_copy(q_hbm.at[dst_v.at[b]],
                                  q_rows.at[b], semq[b]).wait()
            pltpu.make_async_copy(kv_hbm.at[src_v.at[b]],
                                  kv_rows.at[b], semk[b]).wait()
            ebase = (base0 + j + b) * CA
            pltpu.sync_copy(q_rows.at[b], qg_hbm.at[pl.ds(ebase, CA)])
            pltpu.sync_copy(kv_rows.at[b], kvg_hbm.at[pl.ds(ebase, CA)])

    pltpu.make_async_copy(e2_hbm.at[0], e2_v.at[1], semi[1]).wait()
    pltpu.make_async_copy(q_hbm.at[dst_v.at[0]], q_rows.at[0], semq[0]).wait()
    pltpu.make_async_copy(kv_hbm.at[src_v.at[0]], kv_rows.at[0], semk[0]).wait()


def _edge_body(q_ref, kv_ref, et_ref, sg_ref, cs_ref, cv_ref, eb_ref,
               selt_ref, sel_ref, o_ref):
    qf = q_ref[...].astype(jnp.float32)
    kf = kv_ref[:, :DIM].astype(jnp.float32)
    vf = kv_ref[:, DIM:].astype(jnp.float32)
    et = et_ref[...]
    sg = sg_ref[...]
    sidx = jnp.where(sg == -1, 0, jnp.where(sg == 1, 1, 2))
    ci = et * 3 + sidx
    iota24 = lax.broadcasted_iota(jnp.int32, (BE, 3 * R), 1)
    oh = (ci == iota24).astype(jnp.float32)
    csrow = jnp.dot(oh, cs_ref[...], preferred_element_type=jnp.float32)
    cvrow = jnp.dot(oh, cv_ref[...], preferred_element_type=jnp.float32)
    ebrow = jnp.dot(oh, eb_ref[...], preferred_element_type=jnp.float32)
    s8 = jnp.dot(qf * kf * csrow, selt_ref[...],
                 preferred_element_type=jnp.float32)
    ex8 = jnp.exp(s8)
    exb = jnp.dot(ex8, sel_ref[...], preferred_element_type=jnp.float32)
    msg = vf * cvrow * exb
    den8 = ex8 * ebrow
    o_ref[...] = jnp.concatenate(
        [msg, den8, jnp.zeros((BE, ACC_W - DIM - H), jnp.float32)], axis=1)


def _edge_math(qg, kvg, et2d, sg2d, cs24, cv24, eb8, selt, sel8):
    grid = (EP // BE,)
    qspec = pl.BlockSpec((BE, DIM), lambda i: (i, 0))
    kvspec = pl.BlockSpec((BE, 2 * DIM), lambda i: (i, 0))
    ispec = pl.BlockSpec((BE, 1), lambda i: (i, 0))
    cspec = pl.BlockSpec((3 * R, DIM), lambda i: (0, 0))
    ebspec = pl.BlockSpec((3 * R, H), lambda i: (0, 0))
    seltspec = pl.BlockSpec((DIM, H), lambda i: (0, 0))
    selspec = pl.BlockSpec((H, DIM), lambda i: (0, 0))
    ospec = pl.BlockSpec((BE, ACC_W), lambda i: (i, 0))
    return pl.pallas_call(
        _edge_body,
        grid=grid,
        in_specs=[qspec, kvspec, ispec, ispec, cspec, cspec, ebspec,
                  seltspec, selspec],
        out_specs=ospec,
        out_shape=jax.ShapeDtypeStruct((EP, ACC_W), jnp.float32),
    )(qg, kvg, et2d, sg2d, cs24, cv24, eb8, selt, sel8)


@functools.partial(
    pl.kernel,
    mesh=_mesh,
    compiler_params=_sc_params,
    out_type=jax.ShapeDtypeStruct((NC * N, ACC_W), jnp.float32),
    scratch_types=[
        pltpu.VMEM_SHARED((NPAD, ACC_W), jnp.float32),
        pltpu.VMEM((2, CB, ACC_W), jnp.float32),
        pltpu.VMEM((2, CB), jnp.int32),
        pltpu.SemaphoreType.DMA,
        pltpu.SemaphoreType.DMA,
        pltpu.SemaphoreType.DMA,
        pltpu.SemaphoreType.DMA,
    ],
)
def _scatter_kernel(dst_hbm, msg_hbm, out_hbm,
                    acc_sh, msg_v, dst_v, semm0, semm1, semd0, semd1):
    c = lax.axis_index("c")
    s = lax.axis_index("s")
    wid = c * NS + s
    base0 = wid * NCHB
    semm = (semm0, semm1)
    semd = (semd0, semd1)
    zero16 = jnp.zeros((16,), jnp.float32)

    @pl.loop(0, CB)
    def _(i):
        for j in range(ACC_W // 16):
            msg_v[0, i, pl.ds(j * 16, 16)] = zero16

    nz = ROWS_PT // CB
    rem = ROWS_PT - nz * CB

    @pl.loop(0, nz * CB, step=CB)
    def _(i):
        pltpu.sync_copy(msg_v.at[0], acc_sh.at[pl.ds(s * ROWS_PT + i, CB)])

    pltpu.sync_copy(msg_v.at[0, pl.ds(0, rem)],
                    acc_sh.at[pl.ds(s * ROWS_PT + nz * CB, rem)])

    plsc.subcore_barrier()

    def _issue(j, b):
        ebase = (base0 + jnp.minimum(j, NCHB - 1)) * CB
        pltpu.async_copy(msg_hbm.at[pl.ds(ebase, CB)], msg_v.at[b], semm[b])
        pltpu.async_copy(dst_hbm.at[pl.ds(ebase, CB)], dst_v.at[b], semd[b])

    _issue(0, 0)

    @pl.loop(0, NCHB, step=2)
    def _(j):
        for b in range(2):
            pltpu.make_async_copy(msg_hbm.at[pl.ds(0, CB)], msg_v.at[b],
                                  semm[b]).wait()
            pltpu.make_async_copy(dst_hbm.at[pl.ds(0, CB)], dst_v.at[b],
                                  semd[b]).wait()
            _issue(j + b + 1, 1 - b)
            pltpu.sync_copy(msg_v.at[b], acc_sh.at[dst_v.at[b]], add=True)

    pltpu.make_async_copy(msg_hbm.at[pl.ds(0, CB)], msg_v.at[0], semm[0]).wait()
    pltpu.make_async_copy(dst_hbm.at[pl.ds(0, CB)], dst_v.at[0], semd[0]).wait()

    plsc.subcore_barrier()
    pltpu.sync_copy(acc_sh.at[pl.ds(s * ROWS_PT, ROWS_PT)],
                    out_hbm.at[pl.ds(c * N + s * ROWS_PT, ROWS_PT)])


def _final_body(a0_ref, a1_ref, x_ref, t_ref, sel_ref, alpha_ref,
                gamma_ref, beta_ref, o_ref):
    num = a0_ref[:, :DIM] + a1_ref[:, :DIM]
    den8 = a0_ref[:, DIM:DIM + H] + a1_ref[:, DIM:DIM + H]
    den = jnp.dot(den8, sel_ref[...], preferred_element_type=jnp.float32)
    out = num / jnp.maximum(den, 1e-16)
    x = x_ref[...]
    t = t_ref[...]
    iota_t = lax.broadcasted_iota(jnp.int32, (t.shape[0], T), 1)
    onehot = (t == iota_t).astype(jnp.float32)
    arow = jnp.dot(onehot, alpha_ref[...], preferred_element_type=jnp.float32)
    grow = jnp.dot(onehot, gamma_ref[...], preferred_element_type=jnp.float32)
    brow = jnp.dot(onehot, beta_ref[...], preferred_element_type=jnp.float32)
    hm = arow * out + (1.0 - arow) * x
    mu = jnp.mean(hm, axis=1, keepdims=True)
    var = jnp.mean((hm - mu) ** 2, axis=1, keepdims=True)
    o_ref[...] = (hm - mu) * lax.rsqrt(var + 1e-5) * grow + brow


def _finalize(acc0, acc1, node_inp, node_type2d, sel8, alpha_mat, gamma, beta):
    B = 1000
    grid = (N // B,)
    aspec = pl.BlockSpec((B, ACC_W), lambda i: (i, 0))
    row = pl.BlockSpec((B, DIM), lambda i: (i, 0))
    tspec = pl.BlockSpec((B, 1), lambda i: (i, 0))
    sspec = pl.BlockSpec((H, DIM), lambda i: (0, 0))
    pspec = pl.BlockSpec((T, DIM), lambda i: (0, 0))
    return pl.pallas_call(
        _final_body,
        grid=grid,
        in_specs=[aspec, aspec, row, tspec, sspec, pspec, pspec, pspec],
        out_specs=row,
        out_shape=jax.ShapeDtypeStruct((N, DIM), jnp.float32),
    )(acc0, acc1, node_inp, node_type2d, sel8, alpha_mat, gamma, beta)


def kernel(node_inp, node_type, edge_index, edge_type, edge_sign,
           Wq, bq, Wk, bk, Wv, bv, rel_q, rel_k, rel_v,
           sign_k_fixed, sign_v_fixed, sign_k_neutral, sign_v_neutral,
           rel_bias, skip, gamma, beta):
    src = edge_index[0].astype(jnp.int32)
    dst = edge_index[1].astype(jnp.int32)
    et = edge_type.astype(jnp.int32)
    sg = edge_sign.astype(jnp.int32)
    padn = EPX - E
    src = jnp.concatenate([src, jnp.zeros((padn,), jnp.int32)])
    dst = jnp.concatenate([dst, jnp.full((padn,), N, jnp.int32)])
    et = jnp.concatenate([et, jnp.zeros((padn,), jnp.int32)])
    sg = jnp.concatenate([sg, jnp.zeros((padn,), jnp.int32)])
    e2 = jnp.stack([src.reshape(-1, CA), dst.reshape(-1, CA)], axis=1)
    node_type2d = node_type.astype(jnp.int32).reshape(N, 1)

    sk_all = jnp.concatenate([sign_k_fixed, sign_k_neutral[None]], axis=0)
    sv_all = jnp.concatenate([sign_v_fixed, sign_v_neutral[None]], axis=0)
    eb = jnp.exp(rel_bias)
    cs24 = ((rel_q * rel_k)[:, None] * sk_all[None]
            / math.sqrt(DK)).reshape(R * 3, DIM)
    cv24 = (rel_v[:, None] * sv_all[None]
            * eb[:, None, :, None]).reshape(R * 3, DIM)
    eb8 = jnp.tile(eb[:, None], (1, 3, 1)).reshape(R * 3, H)

    alphas = jax.nn.sigmoid(skip)
    alpha_mat = jnp.broadcast_to(alphas[:, None], (T, DIM)).astype(jnp.float32)
    sel8 = jnp.kron(jnp.eye(H, dtype=jnp.float32),
                    jnp.ones((1, DK), jnp.float32))

    q, kv = _project(node_inp, node_type2d, Wq, bq, Wk, bk, Wv, bv)
    qb = jnp.concatenate([q.astype(jnp.bfloat16),
                          jnp.zeros((NPAD - N, DIM), jnp.bfloat16)])
    kvb = jnp.concatenate([kv.astype(jnp.bfloat16),
                           jnp.zeros((NPAD - N, 2 * DIM), jnp.bfloat16)])

    qg, kvg = _gather_kernel(e2, qb, kvb)
    msg = _edge_math(qg, kvg, et[:EP].reshape(EP, 1), sg[:EP].reshape(EP, 1),
                     cs24, cv24, eb8, sel8.T, sel8)
    acc = _scatter_kernel(dst, msg)
    return _finalize(acc[:N], acc[N:], node_inp, node_type2d,
                     sel8, alpha_mat, gamma, beta)


# speedup vs baseline: 1.2897x; 1.2897x over previous
import functools
import math

import jax
import jax.numpy as jnp
from jax import lax
from jax.experimental import pallas as pl
from jax.experimental.pallas import tpu as pltpu
from jax.experimental.pallas import tpu_sc as plsc

N = 10000
E = 320000
DIM = 128
T = 4
R = 8
H = 8
DK = 16

NC = 2
NS = 16
NW = NC * NS
CA = 128
NCHA = 2 * (-(-E // (NW * CA * 2)))
EPW = NCHA * CA
EP = NW * EPW
EPX = (NW * NCHA + 2) * CA
CB = 64
NCHB = EPW // CB
ACC_W = 144
NPAD = N + 16
ROWS_PT = N // NS
BE = 2048

_mesh = plsc.VectorSubcoreMesh(core_axis_name="c", subcore_axis_name="s")
_sc_params = pltpu.CompilerParams(use_tc_tiling_on_sc=False,
                                  needs_layout_passes=False)


def _proj_body(x_ref, t_ref, wq_ref, bq_ref, wk_ref, bk_ref, wv_ref, bv_ref,
               q_ref, k_ref, v_ref):
    x = x_ref[...]
    t = t_ref[...]
    for out_ref, w_ref, b_ref in ((q_ref, wq_ref, bq_ref),
                                  (k_ref, wk_ref, bk_ref),
                                  (v_ref, wv_ref, bv_ref)):
        acc = jnp.zeros(x.shape, jnp.float32)
        for tt in range(T):
            sel = (t == tt).astype(jnp.float32)
            y = jnp.dot(x, w_ref[tt], preferred_element_type=jnp.float32)
            acc = acc + sel * (y + b_ref[tt:tt + 1, :])
        out_ref[...] = acc


def _project(node_inp, node_type2d, Wq, bq, Wk, bk, Wv, bv):
    B = 1000
    grid = (N // B,)
    row = pl.BlockSpec((B, DIM), lambda i: (i, 0))
    tspec = pl.BlockSpec((B, 1), lambda i: (i, 0))
    wspec = pl.BlockSpec((T, DIM, DIM), lambda i: (0, 0, 0))
    bspec = pl.BlockSpec((T, DIM), lambda i: (0, 0))
    out = jax.ShapeDtypeStruct((N, DIM), jnp.float32)
    return pl.pallas_call(
        _proj_body,
        grid=grid,
        in_specs=[row, tspec, wspec, bspec, wspec, bspec, wspec, bspec],
        out_specs=[row, row, row],
        out_shape=[out, out, out],
    )(node_inp, node_type2d, Wq, bq, Wk, bk, Wv, bv)


@functools.partial(
    pl.kernel,
    mesh=_mesh,
    compiler_params=_sc_params,
    out_type=[jax.ShapeDtypeStruct((EP, DIM), jnp.float32),
              jax.ShapeDtypeStruct((EP, DIM), jnp.float32),
              jax.ShapeDtypeStruct((EP, DIM), jnp.float32)],
    scratch_types=[
        pltpu.VMEM((2, 2, CA), jnp.int32),
        pltpu.VMEM((2, CA), jnp.int32),
        pltpu.VMEM((2, CA), jnp.int32),
        pltpu.VMEM((2, CA, DIM), jnp.float32),
        pltpu.VMEM((2, CA, DIM), jnp.float32),
        pltpu.VMEM((2, CA, DIM), jnp.float32),
        pltpu.SemaphoreType.DMA,
        pltpu.SemaphoreType.DMA,
        pltpu.SemaphoreType.DMA,
        pltpu.SemaphoreType.DMA,
        pltpu.SemaphoreType.DMA,
        pltpu.SemaphoreType.DMA,
        pltpu.SemaphoreType.DMA,
        pltpu.SemaphoreType.DMA,
    ],
)
def _gather_kernel(e2_hbm, q_hbm, k_hbm, v_hbm, qg_hbm, kg_hbm, vg_hbm,
                   e2_v, src_v, dst_v, q_rows, k_rows, v_rows,
                   semi0, semi1, semq0, semq1, semk0, semk1, semv0, semv1):
    c = lax.axis_index("c")
    s = lax.axis_index("s")
    wid = c * NS + s
    base0 = wid * NCHA
    semi = (semi0, semi1)
    semq = (semq0, semq1)
    semk = (semk0, semk1)
    semv = (semv0, semv1)

    def _unpack(b):
        for i in range(CA // 16):
            sl = pl.ds(i * 16, 16)
            src_v[b, sl] = e2_v[b, 0, sl]
            dst_v[b, sl] = e2_v[b, 1, sl]

    def _issue_idx(j, b):
        return pltpu.async_copy(e2_hbm.at[pl.ds(2 * (base0 + j), 2)],
                                e2_v.at[b], semi[b])

    def _issue_gathers(b):
        pltpu.async_copy(q_hbm.at[dst_v.at[b]], q_rows.at[b], semq[b])
        pltpu.async_copy(k_hbm.at[src_v.at[b]], k_rows.at[b], semk[b])
        pltpu.async_copy(v_hbm.at[src_v.at[b]], v_rows.at[b], semv[b])

    _issue_idx(0, 0).wait()
    _unpack(0)
    _issue_gathers(0)
    _issue_idx(1, 1)

    @pl.loop(0, NCHA, step=2)
    def _(j):
        for b in range(2):
            pltpu.make_async_copy(e2_hbm.at[pl.ds(0, 2)], e2_v.at[1 - b],
                                  semi[1 - b]).wait()
            _unpack(1 - b)
            _issue_gathers(1 - b)
            _issue_idx(j + b + 2, b)
            pltpu.make_async_copy(q_hbm.at[dst_v.at[b]],
                                  q_rows.at[b], semq[b]).wait()
            pltpu.make_async_copy(k_hbm.at[src_v.at[b]],
                                  k_rows.at[b], semk[b]).wait()
            pltpu.make_async_copy(v_hbm.at[src_v.at[b]],
                                  v_rows.at[b], semv[b]).wait()
            ebase = (base0 + j + b) * CA
            pltpu.sync_copy(q_rows.at[b], qg_hbm.at[pl.ds(ebase, CA)])
            pltpu.sync_copy(k_rows.at[b], kg_hbm.at[pl.ds(ebase, CA)])
            pltpu.sync_copy(v_rows.at[b], vg_hbm.at[pl.ds(ebase, CA)])

    pltpu.make_async_copy(e2_hbm.at[pl.ds(0, 2)], e2_v.at[1], semi[1]).wait()
    pltpu.make_async_copy(q_hbm.at[dst_v.at[0]], q_rows.at[0], semq[0]).wait()
    pltpu.make_async_copy(k_hbm.at[src_v.at[0]], k_rows.at[0], semk[0]).wait()
    pltpu.make_async_copy(v_hbm.at[src_v.at[0]], v_rows.at[0], semv[0]).wait()


def _edge_body(q_ref, k_ref, v_ref, et_ref, sg_ref, cs_ref, cv_ref, eb_ref,
               selt_ref, sel_ref, o_ref):
    qf = q_ref[...]
    kf = k_ref[...]
    vf = v_ref[...]
    et = et_ref[...]
    sg = sg_ref[...]
    sidx = jnp.where(sg == -1, 0, jnp.where(sg == 1, 1, 2))
    ci = et * 3 + sidx
    iota24 = lax.broadcasted_iota(jnp.int32, (BE, 3 * R), 1)
    oh = (ci == iota24).astype(jnp.float32)
    csrow = jnp.dot(oh, cs_ref[...], preferred_element_type=jnp.float32)
    cvrow = jnp.dot(oh, cv_ref[...], preferred_element_type=jnp.float32)
    ebrow = jnp.dot(oh, eb_ref[...], preferred_element_type=jnp.float32)
    s8 = jnp.dot(qf * kf * csrow, selt_ref[...],
                 preferred_element_type=jnp.float32)
    ex8 = jnp.exp(s8)
    exb = jnp.dot(ex8, sel_ref[...], preferred_element_type=jnp.float32)
    msg = vf * cvrow * exb
    den8 = ex8 * ebrow
    o_ref[...] = jnp.concatenate(
        [msg, den8, jnp.zeros((BE, ACC_W - DIM - H), jnp.float32)], axis=1)


def _edge_math(qg, kg, vg, et2d, sg2d, cs24, cv24, eb8, selt, sel8):
    grid = (EP // BE,)
    rspec = pl.BlockSpec((BE, DIM), lambda i: (i, 0))
    ispec = pl.BlockSpec((BE, 1), lambda i: (i, 0))
    cspec = pl.BlockSpec((3 * R, DIM), lambda i: (0, 0))
    ebspec = pl.BlockSpec((3 * R, H), lambda i: (0, 0))
    seltspec = pl.BlockSpec((DIM, H), lambda i: (0, 0))
    selspec = pl.BlockSpec((H, DIM), lambda i: (0, 0))
    ospec = pl.BlockSpec((BE, ACC_W), lambda i: (i, 0))
    return pl.pallas_call(
        _edge_body,
        grid=grid,
        in_specs=[rspec, rspec, rspec, ispec, ispec, cspec, cspec, ebspec,
                  seltspec, selspec],
        out_specs=ospec,
        out_shape=jax.ShapeDtypeStruct((EP, ACC_W), jnp.float32),
    )(qg, kg, vg, et2d, sg2d, cs24, cv24, eb8, selt, sel8)


@functools.partial(
    pl.kernel,
    mesh=_mesh,
    compiler_params=_sc_params,
    out_type=jax.ShapeDtypeStruct((NC * N, ACC_W), jnp.float32),
    scratch_types=[
        pltpu.VMEM_SHARED((NPAD, ACC_W), jnp.float32),
        pltpu.VMEM((2, CB, ACC_W), jnp.float32),
        pltpu.VMEM((2, CB), jnp.int32),
        pltpu.SemaphoreType.DMA,
        pltpu.SemaphoreType.DMA,
        pltpu.SemaphoreType.DMA,
        pltpu.SemaphoreType.DMA,
    ],
)
def _scatter_kernel(dst_hbm, msg_hbm, out_hbm,
                    acc_sh, msg_v, dst_v, semm0, semm1, semd0, semd1):
    c = lax.axis_index("c")
    s = lax.axis_index("s")
    wid = c * NS + s
    base0 = wid * NCHB
    semm = (semm0, semm1)
    semd = (semd0, semd1)
    zero16 = jnp.zeros((16,), jnp.float32)

    @pl.loop(0, CB)
    def _(i):
        for j in range(ACC_W // 16):
            msg_v[0, i, pl.ds(j * 16, 16)] = zero16

    nz = ROWS_PT // CB
    rem = ROWS_PT - nz * CB

    @pl.loop(0, nz * CB, step=CB)
    def _(i):
        pltpu.sync_copy(msg_v.at[0], acc_sh.at[pl.ds(s * ROWS_PT + i, CB)])

    pltpu.sync_copy(msg_v.at[0, pl.ds(0, rem)],
                    acc_sh.at[pl.ds(s * ROWS_PT + nz * CB, rem)])

    plsc.subcore_barrier()

    def _issue(j, b):
        ebase = (base0 + jnp.minimum(j, NCHB - 1)) * CB
        pltpu.async_copy(msg_hbm.at[pl.ds(ebase, CB)], msg_v.at[b], semm[b])
        pltpu.async_copy(dst_hbm.at[pl.ds(ebase, CB)], dst_v.at[b], semd[b])

    _issue(0, 0)

    @pl.loop(0, NCHB, step=2)
    def _(j):
        for b in range(2):
            pltpu.make_async_copy(msg_hbm.at[pl.ds(0, CB)], msg_v.at[b],
                                  semm[b]).wait()
            pltpu.make_async_copy(dst_hbm.at[pl.ds(0, CB)], dst_v.at[b],
                                  semd[b]).wait()
            _issue(j + b + 1, 1 - b)
            pltpu.sync_copy(msg_v.at[b], acc_sh.at[dst_v.at[b]], add=True)

    pltpu.make_async_copy(msg_hbm.at[pl.ds(0, CB)], msg_v.at[0], semm[0]).wait()
    pltpu.make_async_copy(dst_hbm.at[pl.ds(0, CB)], dst_v.at[0], semd[0]).wait()

    plsc.subcore_barrier()
    pltpu.sync_copy(acc_sh.at[pl.ds(s * ROWS_PT, ROWS_PT)],
                    out_hbm.at[pl.ds(c * N + s * ROWS_PT, ROWS_PT)])


def _final_body(a0_ref, a1_ref, x_ref, t_ref, sel_ref, alpha_ref,
                gamma_ref, beta_ref, o_ref):
    num = a0_ref[:, :DIM] + a1_ref[:, :DIM]
    den8 = a0_ref[:, DIM:DIM + H] + a1_ref[:, DIM:DIM + H]
    den = jnp.dot(den8, sel_ref[...], preferred_element_type=jnp.float32)
    out = num / jnp.maximum(den, 1e-16)
    x = x_ref[...]
    t = t_ref[...]
    iota_t = lax.broadcasted_iota(jnp.int32, (t.shape[0], T), 1)
    onehot = (t == iota_t).astype(jnp.float32)
    arow = jnp.dot(onehot, alpha_ref[...], preferred_element_type=jnp.float32)
    grow = jnp.dot(onehot, gamma_ref[...], preferred_element_type=jnp.float32)
    brow = jnp.dot(onehot, beta_ref[...], preferred_element_type=jnp.float32)
    hm = arow * out + (1.0 - arow) * x
    mu = jnp.mean(hm, axis=1, keepdims=True)
    var = jnp.mean((hm - mu) ** 2, axis=1, keepdims=True)
    o_ref[...] = (hm - mu) * lax.rsqrt(var + 1e-5) * grow + brow


def _finalize(acc0, acc1, node_inp, node_type2d, sel8, alpha_mat, gamma, beta):
    B = 1000
    grid = (N // B,)
    aspec = pl.BlockSpec((B, ACC_W), lambda i: (i, 0))
    row = pl.BlockSpec((B, DIM), lambda i: (i, 0))
    tspec = pl.BlockSpec((B, 1), lambda i: (i, 0))
    sspec = pl.BlockSpec((H, DIM), lambda i: (0, 0))
    pspec = pl.BlockSpec((T, DIM), lambda i: (0, 0))
    return pl.pallas_call(
        _final_body,
        grid=grid,
        in_specs=[aspec, aspec, row, tspec, sspec, pspec, pspec, pspec],
        out_specs=row,
        out_shape=jax.ShapeDtypeStruct((N, DIM), jnp.float32),
    )(acc0, acc1, node_inp, node_type2d, sel8, alpha_mat, gamma, beta)


def kernel(node_inp, node_type, edge_index, edge_type, edge_sign,
           Wq, bq, Wk, bk, Wv, bv, rel_q, rel_k, rel_v,
           sign_k_fixed, sign_v_fixed, sign_k_neutral, sign_v_neutral,
           rel_bias, skip, gamma, beta):
    src = edge_index[0].astype(jnp.int32)
    dst = edge_index[1].astype(jnp.int32)
    et = edge_type.astype(jnp.int32)
    sg = edge_sign.astype(jnp.int32)
    padn = EPX - E
    src = jnp.concatenate([src, jnp.zeros((padn,), jnp.int32)])
    dst = jnp.concatenate([dst, jnp.full((padn,), N, jnp.int32)])
    et = jnp.concatenate([et, jnp.zeros((padn,), jnp.int32)])
    sg = jnp.concatenate([sg, jnp.zeros((padn,), jnp.int32)])
    e2 = jnp.stack([src.reshape(-1, CA), dst.reshape(-1, CA)],
                   axis=1).reshape(-1, CA)
    node_type2d = node_type.astype(jnp.int32).reshape(N, 1)

    sk_all = jnp.concatenate([sign_k_fixed, sign_k_neutral[None]], axis=0)
    sv_all = jnp.concatenate([sign_v_fixed, sign_v_neutral[None]], axis=0)
    eb = jnp.exp(rel_bias)
    cs24 = ((rel_q * rel_k)[:, None] * sk_all[None]
            / math.sqrt(DK)).reshape(R * 3, DIM)
    cv24 = (rel_v[:, None] * sv_all[None]
            * eb[:, None, :, None]).reshape(R * 3, DIM)
    eb8 = jnp.tile(eb[:, None], (1, 3, 1)).reshape(R * 3, H)

    alphas = jax.nn.sigmoid(skip)
    alpha_mat = jnp.broadcast_to(alphas[:, None], (T, DIM)).astype(jnp.float32)
    sel8 = jnp.kron(jnp.eye(H, dtype=jnp.float32),
                    jnp.ones((1, DK), jnp.float32))

    q, k, v = _project(node_inp, node_type2d, Wq, bq, Wk, bk, Wv, bv)
    zrows = jnp.zeros((NPAD - N, DIM), jnp.float32)
    q = jnp.concatenate([q, zrows])
    k = jnp.concatenate([k, zrows])
    v = jnp.concatenate([v, zrows])

    qg, kg, vg = _gather_kernel(e2, q, k, v)
    msg = _edge_math(qg, kg, vg, et[:EP].reshape(EP, 1), sg[:EP].reshape(EP, 1),
                     cs24, cv24, eb8, sel8.T, sel8)
    acc = _scatter_kernel(dst, msg)
    return _finalize(acc[:N], acc[N:], node_inp, node_type2d,
                     sel8, alpha_mat, gamma, beta)


# speedup vs baseline: 1.5875x; 1.2309x over previous
import functools
import math

import jax
import jax.numpy as jnp
from jax import lax
from jax.experimental import pallas as pl
from jax.experimental.pallas import tpu as pltpu
from jax.experimental.pallas import tpu_sc as plsc

N = 10000
E = 320000
DIM = 128
T = 4
R = 8
H = 8
DK = 16

NC = 2
NS = 16
NW = NC * NS
CA = 128
NCHA = 2 * (-(-E // (NW * CA * 2)))
EPW = NCHA * CA
EP = NW * EPW
EPX = (NW * NCHA + 2) * CA
CB = 64
NCHB = EPW // CB
ACC_W = 144
NPAD = N + 16
ROWS_PT = N // NS
BE = 2048

_mesh = plsc.VectorSubcoreMesh(core_axis_name="c", subcore_axis_name="s")
_sc_params = pltpu.CompilerParams(use_tc_tiling_on_sc=False,
                                  needs_layout_passes=False)


def _proj_body(x_ref, t_ref, wq_ref, bq_ref, wk_ref, bk_ref, wv_ref, bv_ref,
               q_ref, k_ref, v_ref):
    x = x_ref[...]
    t = t_ref[...]
    for out_ref, w_ref, b_ref in ((q_ref, wq_ref, bq_ref),
                                  (k_ref, wk_ref, bk_ref),
                                  (v_ref, wv_ref, bv_ref)):
        acc = jnp.zeros(x.shape, jnp.float32)
        for tt in range(T):
            sel = (t == tt).astype(jnp.float32)
            y = jnp.dot(x, w_ref[tt], preferred_element_type=jnp.float32)
            acc = acc + sel * (y + b_ref[tt:tt + 1, :])
        out_ref[...] = acc


def _project(node_inp, node_type2d, Wq, bq, Wk, bk, Wv, bv):
    B = 1000
    grid = (N // B,)
    row = pl.BlockSpec((B, DIM), lambda i: (i, 0))
    tspec = pl.BlockSpec((B, 1), lambda i: (i, 0))
    wspec = pl.BlockSpec((T, DIM, DIM), lambda i: (0, 0, 0))
    bspec = pl.BlockSpec((T, DIM), lambda i: (0, 0))
    out = jax.ShapeDtypeStruct((N, DIM), jnp.float32)
    return pl.pallas_call(
        _proj_body,
        grid=grid,
        in_specs=[row, tspec, wspec, bspec, wspec, bspec, wspec, bspec],
        out_specs=[row, row, row],
        out_shape=[out, out, out],
    )(node_inp, node_type2d, Wq, bq, Wk, bk, Wv, bv)


@functools.partial(
    pl.kernel,
    mesh=_mesh,
    compiler_params=_sc_params,
    out_type=[jax.ShapeDtypeStruct((EP, DIM), jnp.float32),
              jax.ShapeDtypeStruct((EP, DIM), jnp.float32),
              jax.ShapeDtypeStruct((EP, DIM), jnp.float32)],
    scratch_types=[
        pltpu.VMEM((2, 2, CA), jnp.int32),
        pltpu.VMEM((2, CA), jnp.int32),
        pltpu.VMEM((2, CA), jnp.int32),
        pltpu.VMEM((2, CA, DIM), jnp.float32),
        pltpu.VMEM((2, CA, DIM), jnp.float32),
        pltpu.VMEM((2, CA, DIM), jnp.float32),
        pltpu.SemaphoreType.DMA,
        pltpu.SemaphoreType.DMA,
        pltpu.SemaphoreType.DMA,
        pltpu.SemaphoreType.DMA,
        pltpu.SemaphoreType.DMA,
        pltpu.SemaphoreType.DMA,
        pltpu.SemaphoreType.DMA,
        pltpu.SemaphoreType.DMA,
    ],
)
def _gather_kernel(e2_hbm, q_hbm, k_hbm, v_hbm, qg_hbm, kg_hbm, vg_hbm,
                   e2_v, src_v, dst_v, q_rows, k_rows, v_rows,
                   semi0, semi1, semq0, semq1, semk0, semk1, semv0, semv1):
    c = lax.axis_index("c")
    s = lax.axis_index("s")
    wid = c * NS + s
    base0 = wid * NCHA
    semi = (semi0, semi1)
    semq = (semq0, semq1)
    semk = (semk0, semk1)
    semv = (semv0, semv1)

    def _unpack(b):
        for i in range(CA // 16):
            sl = pl.ds(i * 16, 16)
            src_v[b, sl] = e2_v[b, 0, sl]
            dst_v[b, sl] = e2_v[b, 1, sl]

    def _issue_idx(j, b):
        return pltpu.async_copy(e2_hbm.at[pl.ds(2 * (base0 + j), 2)],
                                e2_v.at[b], semi[b])

    def _issue_gathers(b):
        pltpu.async_copy(q_hbm.at[dst_v.at[b]], q_rows.at[b], semq[b])
        pltpu.async_copy(k_hbm.at[src_v.at[b]], k_rows.at[b], semk[b])
        pltpu.async_copy(v_hbm.at[src_v.at[b]], v_rows.at[b], semv[b])

    _issue_idx(0, 0).wait()
    _unpack(0)
    _issue_gathers(0)
    _issue_idx(1, 1)

    @pl.loop(0, NCHA, step=2)
    def _(j):
        for b in range(2):
            pltpu.make_async_copy(e2_hbm.at[pl.ds(0, 2)], e2_v.at[1 - b],
                                  semi[1 - b]).wait()
            _unpack(1 - b)
            _issue_gathers(1 - b)
            _issue_idx(j + b + 2, b)
            pltpu.make_async_copy(q_hbm.at[dst_v.at[b]],
                                  q_rows.at[b], semq[b]).wait()
            pltpu.make_async_copy(k_hbm.at[src_v.at[b]],
                                  k_rows.at[b], semk[b]).wait()
            pltpu.make_async_copy(v_hbm.at[src_v.at[b]],
                                  v_rows.at[b], semv[b]).wait()
            ebase = (base0 + j + b) * CA
            pltpu.sync_copy(q_rows.at[b], qg_hbm.at[pl.ds(ebase, CA)])
            pltpu.sync_copy(k_rows.at[b], kg_hbm.at[pl.ds(ebase, CA)])
            pltpu.sync_copy(v_rows.at[b], vg_hbm.at[pl.ds(ebase, CA)])

    pltpu.make_async_copy(e2_hbm.at[pl.ds(0, 2)], e2_v.at[1], semi[1]).wait()
    pltpu.make_async_copy(q_hbm.at[dst_v.at[0]], q_rows.at[0], semq[0]).wait()
    pltpu.make_async_copy(k_hbm.at[src_v.at[0]], k_rows.at[0], semk[0]).wait()
    pltpu.make_async_copy(v_hbm.at[src_v.at[0]], v_rows.at[0], semv[0]).wait()


def _edge_body(q_ref, k_ref, v_ref, et_ref, sg_ref, cs_ref, cv_ref, eb_ref,
               selt_ref, sel_ref, o_ref, d_ref):
    qf = q_ref[...]
    kf = k_ref[...]
    vf = v_ref[...]
    et = et_ref[...]
    sg = sg_ref[...]
    sidx = jnp.where(sg == -1, 0, jnp.where(sg == 1, 1, 2))
    ci = et * 3 + sidx
    iota24 = lax.broadcasted_iota(jnp.int32, (BE, 3 * R), 1)
    oh = (ci == iota24).astype(jnp.float32)
    csrow = jnp.dot(oh, cs_ref[...], preferred_element_type=jnp.float32)
    cvrow = jnp.dot(oh, cv_ref[...], preferred_element_type=jnp.float32)
    ebrow = jnp.dot(oh, eb_ref[...], preferred_element_type=jnp.float32)
    s8 = jnp.dot(qf * kf * csrow, selt_ref[...],
                 preferred_element_type=jnp.float32)
    ex8 = jnp.exp(s8)
    exb = jnp.dot(ex8, sel_ref[...], preferred_element_type=jnp.float32)
    o_ref[...] = vf * cvrow * exb
    den8 = ex8 * ebrow
    d_ref[...] = jnp.concatenate(
        [den8, jnp.zeros((BE, DIM - H), jnp.float32)], axis=1)


def _edge_math(qg, kg, vg, et2d, sg2d, cs24, cv24, eb8, selt, sel8):
    grid = (EP // BE,)
    rspec = pl.BlockSpec((BE, DIM), lambda i: (i, 0))
    ispec = pl.BlockSpec((BE, 1), lambda i: (i, 0))
    cspec = pl.BlockSpec((3 * R, DIM), lambda i: (0, 0))
    ebspec = pl.BlockSpec((3 * R, H), lambda i: (0, 0))
    seltspec = pl.BlockSpec((DIM, H), lambda i: (0, 0))
    selspec = pl.BlockSpec((H, DIM), lambda i: (0, 0))
    return pl.pallas_call(
        _edge_body,
        grid=grid,
        in_specs=[rspec, rspec, rspec, ispec, ispec, cspec, cspec, ebspec,
                  seltspec, selspec],
        out_specs=[rspec, rspec],
        out_shape=[jax.ShapeDtypeStruct((EP, DIM), jnp.float32),
                   jax.ShapeDtypeStruct((EP, DIM), jnp.float32)],
    )(qg, kg, vg, et2d, sg2d, cs24, cv24, eb8, selt, sel8)


@functools.partial(
    pl.kernel,
    mesh=_mesh,
    compiler_params=_sc_params,
    out_type=[jax.ShapeDtypeStruct((NC * N, DIM), jnp.float32),
              jax.ShapeDtypeStruct((NC * N, 16), jnp.float32)],
    scratch_types=[
        pltpu.VMEM_SHARED((NPAD, DIM), jnp.float32),
        pltpu.VMEM_SHARED((NPAD, 16), jnp.float32),
        pltpu.VMEM((2, CB, DIM), jnp.float32),
        pltpu.VMEM((2, CB, DIM), jnp.float32),
        pltpu.VMEM((2, CB, 16), jnp.float32),
        pltpu.VMEM((2, CB), jnp.int32),
        pltpu.SemaphoreType.DMA,
        pltpu.SemaphoreType.DMA,
        pltpu.SemaphoreType.DMA,
        pltpu.SemaphoreType.DMA,
        pltpu.SemaphoreType.DMA,
        pltpu.SemaphoreType.DMA,
    ],
)
def _scatter_kernel(dst_hbm, msg_hbm, den_hbm, outm_hbm, outd_hbm,
                    accm_sh, accd_sh, msg_v, den_v, den16_v, dst_v,
                    semm0, semm1, semn0, semn1, semd0, semd1):
    c = lax.axis_index("c")
    s = lax.axis_index("s")
    wid = c * NS + s
    base0 = wid * NCHB
    semm = (semm0, semm1)
    semn = (semn0, semn1)
    semd = (semd0, semd1)
    zero16 = jnp.zeros((16,), jnp.float32)

    @pl.loop(0, CB)
    def _(i):
        for j in range(DIM // 16):
            msg_v[0, i, pl.ds(j * 16, 16)] = zero16
        den16_v[0, i, pl.ds(0, 16)] = zero16

    nz = ROWS_PT // CB
    rem = ROWS_PT - nz * CB

    @pl.loop(0, nz * CB, step=CB)
    def _(i):
        pltpu.sync_copy(msg_v.at[0], accm_sh.at[pl.ds(s * ROWS_PT + i, CB)])
        pltpu.sync_copy(den16_v.at[0], accd_sh.at[pl.ds(s * ROWS_PT + i, CB)])

    pltpu.sync_copy(msg_v.at[0, pl.ds(0, rem)],
                    accm_sh.at[pl.ds(s * ROWS_PT + nz * CB, rem)])
    pltpu.sync_copy(den16_v.at[0, pl.ds(0, rem)],
                    accd_sh.at[pl.ds(s * ROWS_PT + nz * CB, rem)])

    plsc.subcore_barrier()

    def _issue(j, b):
        jj = jnp.minimum(j, NCHB - 1)
        ebase = (base0 + jj) * CB
        pltpu.async_copy(msg_hbm.at[pl.ds(ebase, CB)], msg_v.at[b], semm[b])
        pltpu.async_copy(den_hbm.at[pl.ds(ebase, CB)], den_v.at[b], semn[b])
        pltpu.async_copy(dst_hbm.at[pl.ds(ebase, CB)], dst_v.at[b], semd[b])

    _issue(0, 0)

    @pl.loop(0, NCHB, step=2)
    def _(j):
        for b in range(2):
            pltpu.make_async_copy(msg_hbm.at[pl.ds(0, CB)], msg_v.at[b],
                                  semm[b]).wait()
            pltpu.make_async_copy(den_hbm.at[pl.ds(0, CB)], den_v.at[b],
                                  semn[b]).wait()
            pltpu.make_async_copy(dst_hbm.at[pl.ds(0, CB)], dst_v.at[b],
                                  semd[b]).wait()
            _issue(j + b + 1, 1 - b)
            for e in range(CB):
                den16_v[b, e, pl.ds(0, 16)] = den_v[b, e, pl.ds(0, 16)]
            pltpu.sync_copy(msg_v.at[b], accm_sh.at[dst_v.at[b]], add=True)
            pltpu.sync_copy(den16_v.at[b], accd_sh.at[dst_v.at[b]], add=True)

    pltpu.make_async_copy(msg_hbm.at[pl.ds(0, CB)], msg_v.at[0], semm[0]).wait()
    pltpu.make_async_copy(den_hbm.at[pl.ds(0, CB)], den_v.at[0], semn[0]).wait()
    pltpu.make_async_copy(dst_hbm.at[pl.ds(0, CB)], dst_v.at[0], semd[0]).wait()

    plsc.subcore_barrier()
    pltpu.sync_copy(accm_sh.at[pl.ds(s * ROWS_PT, ROWS_PT)],
                    outm_hbm.at[pl.ds(c * N + s * ROWS_PT, ROWS_PT)])
    pltpu.sync_copy(accd_sh.at[pl.ds(s * ROWS_PT, ROWS_PT)],
                    outd_hbm.at[pl.ds(c * N + s * ROWS_PT, ROWS_PT)])


def _final_body(a0_ref, a1_ref, d0_ref, d1_ref, x_ref, t_ref, sel_ref,
                alpha_ref, gamma_ref, beta_ref, o_ref):
    num = a0_ref[...] + a1_ref[...]
    den8 = d0_ref[:, :H] + d1_ref[:, :H]
    den = jnp.dot(den8, sel_ref[...], preferred_element_type=jnp.float32)
    out = num / jnp.maximum(den, 1e-16)
    x = x_ref[...]
    t = t_ref[...]
    iota_t = lax.broadcasted_iota(jnp.int32, (t.shape[0], T), 1)
    onehot = (t == iota_t).astype(jnp.float32)
    arow = jnp.dot(onehot, alpha_ref[...], preferred_element_type=jnp.float32)
    grow = jnp.dot(onehot, gamma_ref[...], preferred_element_type=jnp.float32)
    brow = jnp.dot(onehot, beta_ref[...], preferred_element_type=jnp.float32)
    hm = arow * out + (1.0 - arow) * x
    mu = jnp.mean(hm, axis=1, keepdims=True)
    var = jnp.mean((hm - mu) ** 2, axis=1, keepdims=True)
    o_ref[...] = (hm - mu) * lax.rsqrt(var + 1e-5) * grow + brow


def _finalize(a0, a1, d0, d1, node_inp, node_type2d, sel8, alpha_mat,
              gamma, beta):
    B = 1000
    grid = (N // B,)
    row = pl.BlockSpec((B, DIM), lambda i: (i, 0))
    dspec = pl.BlockSpec((B, 16), lambda i: (i, 0))
    tspec = pl.BlockSpec((B, 1), lambda i: (i, 0))
    sspec = pl.BlockSpec((H, DIM), lambda i: (0, 0))
    pspec = pl.BlockSpec((T, DIM), lambda i: (0, 0))
    return pl.pallas_call(
        _final_body,
        grid=grid,
        in_specs=[row, row, dspec, dspec, row, tspec, sspec, pspec, pspec,
                  pspec],
        out_specs=row,
        out_shape=jax.ShapeDtypeStruct((N, DIM), jnp.float32),
    )(a0, a1, d0, d1, node_inp, node_type2d, sel8, alpha_mat, gamma, beta)


def kernel(node_inp, node_type, edge_index, edge_type, edge_sign,
           Wq, bq, Wk, bk, Wv, bv, rel_q, rel_k, rel_v,
           sign_k_fixed, sign_v_fixed, sign_k_neutral, sign_v_neutral,
           rel_bias, skip, gamma, beta):
    src = edge_index[0].astype(jnp.int32)
    dst = edge_index[1].astype(jnp.int32)
    et = edge_type.astype(jnp.int32)
    sg = edge_sign.astype(jnp.int32)
    padn = EPX - E
    src = jnp.concatenate([src, jnp.zeros((padn,), jnp.int32)])
    dst = jnp.concatenate([dst, jnp.full((padn,), N, jnp.int32)])
    et = jnp.concatenate([et, jnp.zeros((padn,), jnp.int32)])
    sg = jnp.concatenate([sg, jnp.zeros((padn,), jnp.int32)])
    e2 = jnp.stack([src.reshape(-1, CA), dst.reshape(-1, CA)],
                   axis=1).reshape(-1, CA)
    node_type2d = node_type.astype(jnp.int32).reshape(N, 1)

    sk_all = jnp.concatenate([sign_k_fixed, sign_k_neutral[None]], axis=0)
    sv_all = jnp.concatenate([sign_v_fixed, sign_v_neutral[None]], axis=0)
    eb = jnp.exp(rel_bias)
    cs24 = ((rel_q * rel_k)[:, None] * sk_all[None]
            / math.sqrt(DK)).reshape(R * 3, DIM)
    cv24 = (rel_v[:, None] * sv_all[None]
            * eb[:, None, :, None]).reshape(R * 3, DIM)
    eb8 = jnp.tile(eb[:, None], (1, 3, 1)).reshape(R * 3, H)

    alphas = jax.nn.sigmoid(skip)
    alpha_mat = jnp.broadcast_to(alphas[:, None], (T, DIM)).astype(jnp.float32)
    sel8 = jnp.kron(jnp.eye(H, dtype=jnp.float32),
                    jnp.ones((1, DK), jnp.float32))

    q, k, v = _project(node_inp, node_type2d, Wq, bq, Wk, bk, Wv, bv)
    zrows = jnp.zeros((NPAD - N, DIM), jnp.float32)
    q = jnp.concatenate([q, zrows])
    k = jnp.concatenate([k, zrows])
    v = jnp.concatenate([v, zrows])

    qg, kg, vg = _gather_kernel(e2, q, k, v)
    msg, den = _edge_math(qg, kg, vg, et[:EP].reshape(EP, 1),
                          sg[:EP].reshape(EP, 1),
                          cs24, cv24, eb8, sel8.T, sel8)
    accm, accd = _scatter_kernel(dst, msg, den)
    return _finalize(accm[:N], accm[N:], accd[:N], accd[N:],
                     node_inp, node_type2d, sel8, alpha_mat, gamma, beta)
